# baseline (speedup 1.0000x reference)
import functools

import jax
import jax.numpy as jnp
from jax import lax
from jax.experimental import pallas as pl
from jax.experimental.pallas import tpu as pltpu
from jax.experimental.pallas import tpu_sc as plsc

NC = 2
NS = 16
NW = NC * NS
FP = 32
KB = 8
NBUF = 3


def _make_gather(bpw: int, d_phys: int):
    mesh = plsc.VectorSubcoreMesh(
        core_axis_name="c", subcore_axis_name="s", num_cores=NC, num_subcores=NS
    )
    ipw = bpw * FP
    cw = KB * FP
    nch = bpw // KB

    @functools.partial(
        pl.kernel,
        out_type=jax.ShapeDtypeStruct((NW * ipw, d_phys), jnp.float32),
        mesh=mesh,
        compiler_params=pltpu.CompilerParams(
            use_tc_tiling_on_sc=True, needs_layout_passes=False
        ),
        scratch_types=[
            pltpu.VMEM((ipw,), jnp.int32),
            [pltpu.VMEM((cw, d_phys), jnp.float32) for _ in range(NBUF)],
            [pltpu.SemaphoreType.DMA for _ in range(NBUF)],
        ],
    )
    def emb(idx_hbm, table_hbm, out_hbm, idx_v, rows, gsem):
        wid = lax.axis_index("s") * NC + lax.axis_index("c")
        base = wid * ipw
        pltpu.sync_copy(idx_hbm.at[pl.ds(base, ipw)], idx_v)

        for b in range(NBUF):
            pltpu.async_copy(
                table_hbm.at[idx_v.at[pl.ds(b * cw, cw)]], rows[b], gsem[b]
            )

        def body(i, carry):
            for b in range(NBUF):
                k = NBUF * i + b
                pltpu.make_async_copy(
                    table_hbm.at[idx_v.at[pl.ds(0, cw)]], rows[b], gsem[b]
                ).wait()
                pltpu.sync_copy(rows[b], out_hbm.at[pl.ds(base + k * cw, cw)])
                pltpu.async_copy(
                    table_hbm.at[idx_v.at[pl.ds((k + NBUF) * cw, cw)]],
                    rows[b],
                    gsem[b],
                )
            return carry

        lax.fori_loop(0, nch // NBUF - 1, body, 0)

        for b in range(NBUF):
            k = nch - NBUF + b
            pltpu.make_async_copy(
                table_hbm.at[idx_v.at[pl.ds(0, cw)]], rows[b], gsem[b]
            ).wait()
            pltpu.sync_copy(rows[b], out_hbm.at[pl.ds(base + k * cw, cw)])

    return emb


def kernel(obs, table):
    batch, fields = obs.shape
    num_in, d = table.shape
    obs = obs.astype(jnp.int32)

    unit = NW * KB * NBUF
    pad_batch = ((batch + unit - 1) // unit) * unit
    idx = jnp.pad(obs, ((0, pad_batch - batch), (0, FP - fields))).reshape(-1)

    d_phys = 128
    table_p = jnp.pad(table, ((0, 0), (0, d_phys - d)))
    out = _make_gather(pad_batch // NW, d_phys)(idx, table_p)
    out = out.reshape(pad_batch, FP, d_phys)[:batch, :fields, :d]
    return out


# speedup vs baseline: 5.8772x; 5.8772x over previous
import functools

import jax
import jax.numpy as jnp
from jax import lax
from jax.experimental import pallas as pl
from jax.experimental.pallas import tpu as pltpu
from jax.experimental.pallas import tpu_sc as plsc

NC = 2
NS = 16
NW = NC * NS
KB = 16
NBUF = 3


def _make_gather(bpw: int, fields: int, d: int):
    mesh = plsc.VectorSubcoreMesh(
        core_axis_name="c", subcore_axis_name="s", num_cores=NC, num_subcores=NS
    )
    ipw = bpw * fields
    cw = KB * fields
    nch = bpw // KB

    @functools.partial(
        pl.kernel,
        out_type=jax.ShapeDtypeStruct((NW * bpw, fields, d), jnp.float32),
        mesh=mesh,
        compiler_params=pltpu.CompilerParams(
            use_tc_tiling_on_sc=False, needs_layout_passes=False
        ),
        scratch_types=[
            pltpu.VMEM((bpw, fields), jnp.int32),
            pltpu.VMEM((ipw,), jnp.int32),
            [pltpu.VMEM((cw, d), jnp.float32) for _ in range(NBUF)],
            [pltpu.SemaphoreType.DMA for _ in range(NBUF)],
            [pltpu.SemaphoreType.DMA for _ in range(NBUF)],
        ],
    )
    def emb(obs_hbm, table_hbm, out_hbm, obs_v, idx_v, rows, gsem, ssem):
        wid = lax.axis_index("s") * NC + lax.axis_index("c")
        bb0 = wid * bpw
        pltpu.sync_copy(obs_hbm.at[pl.ds(bb0, bpw)], obs_v)

        lanes = lax.broadcasted_iota(jnp.int32, (16,), 0)
        zeros = jnp.zeros((16,), jnp.int32)

        def repack(i, carry):
            r, c = carry
            idx_v[pl.ds(i * 16, 16)] = plsc.load_gather(obs_v, [r, c])
            c2 = c + 16
            wrap = c2 >= fields
            c2 = jnp.where(wrap, c2 - fields, c2)
            r2 = jnp.where(wrap, r + 1, r)
            return r2, c2

        lax.fori_loop(0, ipw // 16, repack, (zeros, lanes))

        def issue(k, slot):
            pltpu.async_copy(
                table_hbm.at[idx_v.at[pl.ds(k * cw, cw)]], rows[slot], gsem[slot]
            )

        def wait_gather(slot):
            pltpu.make_async_copy(
                table_hbm.at[idx_v.at[pl.ds(0, cw)]], rows[slot], gsem[slot]
            ).wait()

        def store_chunk(k, slot):
            for j in range(KB):
                pltpu.async_copy(
                    rows[slot].at[pl.ds(j * fields, fields)],
                    out_hbm.at[bb0 + k * KB + j],
                    ssem[slot],
                )

        def drain_stores(slot):
            pltpu.make_async_copy(
                table_hbm.at[pl.ds(0, cw)], rows[slot], ssem[slot]
            ).wait()

        issue(0, 0)
        issue(1, 1)

        wait_gather(0)
        store_chunk(0, 0)
        issue(2, 2)

        def body(i, carry):
            for u in range(NBUF):
                k = 1 + NBUF * i + u
                slot = (1 + u) % NBUF
                wait_gather(slot)
                store_chunk(k, slot)
                nslot = u
                drain_stores(nslot)
                issue(k + 2, nslot)
            return carry

        nsteady = nch - 5
        lax.fori_loop(0, nsteady // NBUF, body, 0)

        for k in range(nch - 4, nch):
            slot = k % NBUF
            wait_gather(slot)
            store_chunk(k, slot)
            if k + 2 < nch:
                nslot = (k + 2) % NBUF
                drain_stores(nslot)
                issue(k + 2, nslot)

        for k in range(nch - NBUF, nch):
            drain_stores(k % NBUF)

    return emb


def kernel(obs, table):
    batch, fields = obs.shape
    num_in, d = table.shape
    obs = obs.astype(jnp.int32)

    unit = NW * KB
    pad_batch = ((batch + unit - 1) // unit) * unit
    if pad_batch != batch:
        obs = jnp.concatenate(
            [obs, jnp.zeros((pad_batch - batch, fields), dtype=jnp.int32)]
        )

    out = _make_gather(pad_batch // NW, fields, d)(obs, table)
    if pad_batch != batch:
        out = out[:batch]
    return out


# speedup vs baseline: 5.9361x; 1.0100x over previous
import functools

import jax
import jax.numpy as jnp
from jax import lax
from jax.experimental import pallas as pl
from jax.experimental.pallas import tpu as pltpu
from jax.experimental.pallas import tpu_sc as plsc

NC = 2
NS = 16
NW = NC * NS
C = 416
NBUF = 4


def _make_gather(nchunk: int, d: int):
    mesh = plsc.VectorSubcoreMesh(
        core_axis_name="c", subcore_axis_name="s", num_cores=NC, num_subcores=NS
    )
    b_per_w = nchunk * C
    total = NW * b_per_w

    @functools.partial(
        pl.kernel,
        out_type=jax.ShapeDtypeStruct((total, d), jnp.float32),
        mesh=mesh,
        compiler_params=pltpu.CompilerParams(use_tc_tiling_on_sc=False),
        scratch_types=[
            pltpu.VMEM((b_per_w,), jnp.int32),
            [pltpu.VMEM((C, d), jnp.float32) for _ in range(NBUF)],
            [pltpu.SemaphoreType.DMA for _ in range(NBUF)],
        ],
    )
    def emb(idx_hbm, table_hbm, out_hbm, idx_v, rows, gsem):
        wid = lax.axis_index("s") * NC + lax.axis_index("c")
        base = wid * b_per_w
        pltpu.sync_copy(idx_hbm.at[wid], idx_v)

        for b in range(NBUF):
            pltpu.async_copy(
                table_hbm.at[idx_v.at[pl.ds(b * C, C)]], rows[b], gsem[b]
            )

        def body(i, carry):
            for b in range(NBUF):
                jj = NBUF * i + b
                pltpu.make_async_copy(
                    table_hbm.at[idx_v.at[pl.ds(0, C)]], rows[b], gsem[b]
                ).wait()
                pltpu.sync_copy(rows[b], out_hbm.at[pl.ds(base + jj * C, C)])
                pltpu.async_copy(
                    table_hbm.at[idx_v.at[pl.ds((jj + NBUF) * C, C)]],
                    rows[b],
                    gsem[b],
                )
            return carry

        lax.fori_loop(0, nchunk // NBUF - 1, body, 0)

        for b in range(NBUF):
            jj = nchunk - NBUF + b
            pltpu.make_async_copy(
                table_hbm.at[idx_v.at[pl.ds(0, C)]], rows[b], gsem[b]
            ).wait()
            pltpu.sync_copy(rows[b], out_hbm.at[pl.ds(base + jj * C, C)])

    return emb


def kernel(obs, table):
    batch, fields = obs.shape
    num_in, d = table.shape
    total = batch * fields
    flat = obs.reshape(total).astype(jnp.int32)

    stride = NW * C * NBUF
    padded = ((total + stride - 1) // stride) * stride
    if padded != total:
        flat = jnp.concatenate(
            [flat, jnp.zeros((padded - total,), dtype=jnp.int32)]
        )
    nchunk = padded // (NW * C)
    idx = flat.reshape(NW, nchunk * C)

    out = _make_gather(nchunk, d)(idx, table)
    if padded != total:
        out = out[:total]
    return out.reshape(batch, fields, d)
